# baseline (speedup 1.0000x reference)
import functools

import jax
import jax.numpy as jnp
from jax import lax
from jax.experimental import pallas as pl
from jax.experimental.pallas import tpu as pltpu
from jax.experimental.pallas import tpu_sc as plsc

_LANES = 16


def _build_sc_kernel(B, L, E, CL):
    n_bt = B // 128
    n_lg = 32 // n_bt
    l_per_w = L // n_lg
    n_chunks = l_per_w // CL
    mesh = plsc.VectorSubcoreMesh(core_axis_name="c", subcore_axis_name="s")
    num_cores = plsc.get_sparse_core_info().num_cores

    @functools.partial(
        pl.kernel,
        mesh=mesh,
        out_type=jax.ShapeDtypeStruct((L, E // 8, B // 128, 8, 128),
                                      jnp.float32),
        compiler_params=pltpu.CompilerParams(use_tc_tiling_on_sc=False,
                                             needs_layout_passes=False),
        scratch_types=[
            pltpu.VMEM((2, CL, 128), jnp.int32),
            pltpu.VMEM((2, CL, 128), jnp.int32),
            pltpu.VMEM((2, CL, 128, E), jnp.float32),
            pltpu.VMEM((CL, E // 8, 8, 128), jnp.float32),
            pltpu.SemaphoreType.DMA,
            pltpu.SemaphoreType.DMA,
            pltpu.SemaphoreType.DMA,
            pltpu.SemaphoreType.DMA,
        ],
    )
    def sc_kernel(seq_hbm, lab_hbm, tok_hbm, ps_hbm, out_hbm,
                  idx_v, psi_v, rows_v, tile_v, sem_t0, sem_p0, sem_t1, sem_p1):
        wid = lax.axis_index("s") * num_cores + lax.axis_index("c")
        bt = wid % n_bt
        lg = wid // n_bt
        col0 = bt * 128
        sem_t = (sem_t0, sem_t1)
        sem_p = (sem_p0, sem_p1)

        def prep(c):
            s = c % 2
            l0 = lg * l_per_w + c * CL
            pltpu.sync_copy(
                seq_hbm.at[pl.ds(l0, CL), pl.ds(col0, 128)], idx_v.at[s])
            pltpu.sync_copy(
                lab_hbm.at[pl.ds(l0, CL), pl.ds(col0, 128)], psi_v.at[s])
            for li in range(CL):
                for bg in range(128 // _LANES):
                    sl = pl.ds(bg * _LANES, _LANES)
                    psi_v[s, li, sl] = psi_v[s, li, sl] + 2 * (l0 + li)
            return [
                pltpu.async_copy(tok_hbm.at[idx_v.at[s, li]],
                                 rows_v.at[s, li], sem_t[s])
                for li in range(CL)
            ]

        pend = prep(0)
        for c in range(n_chunks):
            s = c % 2
            l0 = lg * l_per_w + c * CL
            for cp in pend:
                cp.wait()
            cps = [
                pltpu.async_copy(ps_hbm.at[psi_v.at[s, li]],
                                 rows_v.at[s, li], sem_p[s], add=True)
                for li in range(CL)
            ]
            nxt = prep(c + 1) if c + 1 < n_chunks else None
            for cp in cps:
                cp.wait()

            def col_body(e, _):
                e3 = e >> 3
                e7 = e & 7
                ecast = jnp.full((_LANES,), e, dtype=jnp.int32)
                for li in range(CL):
                    for bg in range(128 // _LANES):
                        b16 = jnp.arange(_LANES, dtype=jnp.int32) + bg * _LANES
                        val = plsc.load_gather(rows_v.at[s, li], [b16, ecast])
                        tile_v[li, e3, e7, pl.ds(bg * _LANES, _LANES)] = val
                return 0

            lax.fori_loop(0, E, col_body, 0)
            for li in range(CL):
                pltpu.sync_copy(tile_v.at[li], out_hbm.at[l0 + li, :, bt])
            pend = nxt

    return sc_kernel


def kernel(sequence, label, token_table, position_table, segment_table):
    B, L = sequence.shape
    V, E = token_table.shape
    ps_table = (position_table[:L, None, :]
                + segment_table[None, :2, :]).reshape(2 * L, E)
    CL = 5
    sc = _build_sc_kernel(B, L, E, CL)
    out5 = sc(sequence.T, label.T, token_table, ps_table)
    return out5.transpose((2, 4, 0, 1, 3)).reshape(B, L, E)


# speedup vs baseline: 1.0615x; 1.0615x over previous
import functools

import jax
import jax.numpy as jnp
from jax import lax
from jax.experimental import pallas as pl
from jax.experimental.pallas import tpu as pltpu
from jax.experimental.pallas import tpu_sc as plsc

_LANES = 16


def _build_sc_kernel(B, L, E, CL):
    n_bt = B // 128
    n_lg = 32 // n_bt
    l_per_w = L // n_lg
    n_chunks = l_per_w // CL
    mesh = plsc.VectorSubcoreMesh(core_axis_name="c", subcore_axis_name="s")
    num_cores = plsc.get_sparse_core_info().num_cores

    @functools.partial(
        pl.kernel,
        mesh=mesh,
        out_type=jax.ShapeDtypeStruct((L, E // 8, B // 128, 8, 128),
                                      jnp.float32),
        compiler_params=pltpu.CompilerParams(use_tc_tiling_on_sc=False,
                                             needs_layout_passes=False),
        scratch_types=[
            pltpu.VMEM((2, CL, 128), jnp.int32),
            pltpu.VMEM((2, CL, 128), jnp.int32),
            pltpu.VMEM((2, CL, 128, E), jnp.float32),
            pltpu.VMEM((CL, E // 8, 8, 128), jnp.float32),
            pltpu.SemaphoreType.DMA,
            pltpu.SemaphoreType.DMA,
            pltpu.SemaphoreType.DMA,
            pltpu.SemaphoreType.DMA,
        ],
    )
    def sc_kernel(seq_hbm, lab_hbm, tok_hbm, ps_hbm, out_hbm,
                  idx_v, psi_v, rows_v, tile_v, sem_t0, sem_p0, sem_t1, sem_p1):
        wid = lax.axis_index("s") * num_cores + lax.axis_index("c")
        bt = wid % n_bt
        lg = wid // n_bt
        col0 = bt * 128
        sem_t = (sem_t0, sem_t1)
        sem_p = (sem_p0, sem_p1)

        def prep(c):
            s = c % 2
            l0 = lg * l_per_w + c * CL
            pltpu.sync_copy(
                seq_hbm.at[pl.ds(l0, CL), pl.ds(col0, 128)], idx_v.at[s])
            pltpu.sync_copy(
                lab_hbm.at[pl.ds(l0, CL), pl.ds(col0, 128)], psi_v.at[s])
            for li in range(CL):
                for bg in range(128 // _LANES):
                    sl = pl.ds(bg * _LANES, _LANES)
                    psi_v[s, li, sl] = psi_v[s, li, sl] + 2 * (l0 + li)
            return [
                pltpu.async_copy(tok_hbm.at[idx_v.at[s, li]],
                                 rows_v.at[s, li], sem_t[s])
                for li in range(CL)
            ]

        pend = prep(0)
        for c in range(n_chunks):
            s = c % 2
            l0 = lg * l_per_w + c * CL
            for cp in pend:
                cp.wait()
            cps = [
                pltpu.async_copy(ps_hbm.at[psi_v.at[s, li]],
                                 rows_v.at[s, li], sem_p[s], add=True)
                for li in range(CL)
            ]
            nxt = prep(c + 1) if c + 1 < n_chunks else None
            for cp in cps:
                cp.wait()

            b16s = [jnp.arange(_LANES, dtype=jnp.int32) + bg * _LANES
                    for bg in range(128 // _LANES)]

            def col_body(e, _):
                e3 = e >> 3
                e7 = e & 7
                ecast = jnp.full((_LANES,), e, dtype=jnp.int32)
                for li in range(CL):
                    vals = [plsc.load_gather(rows_v.at[s, li], [b16, ecast])
                            for b16 in b16s]
                    for bg, val in enumerate(vals):
                        tile_v[li, e3, e7, pl.ds(bg * _LANES, _LANES)] = val
                return 0

            lax.fori_loop(0, E, col_body, 0)
            for li in range(CL):
                pltpu.sync_copy(tile_v.at[li], out_hbm.at[l0 + li, :, bt])
            pend = nxt

    return sc_kernel


def kernel(sequence, label, token_table, position_table, segment_table):
    B, L = sequence.shape
    V, E = token_table.shape
    ps_table = (position_table[:L, None, :]
                + segment_table[None, :2, :]).reshape(2 * L, E)
    CL = 5
    sc = _build_sc_kernel(B, L, E, CL)
    out5 = sc(sequence.T, label.T, token_table, ps_table)
    return out5.transpose((2, 4, 0, 1, 3)).reshape(B, L, E)


# speedup vs baseline: 1.1297x; 1.0642x over previous
import functools

import jax
import jax.numpy as jnp
from jax import lax
from jax.experimental import pallas as pl
from jax.experimental.pallas import tpu as pltpu
from jax.experimental.pallas import tpu_sc as plsc

_LANES = 16


def _build_sc_kernel(B, L, E, CL):
    n_bt = B // 128
    n_lg = 32 // n_bt
    l_per_w = L // n_lg
    n_chunks = l_per_w // CL
    mesh = plsc.VectorSubcoreMesh(core_axis_name="c", subcore_axis_name="s")
    num_cores = plsc.get_sparse_core_info().num_cores

    @functools.partial(
        pl.kernel,
        mesh=mesh,
        out_type=jax.ShapeDtypeStruct((L, E // 8, B // 128, 8, 128),
                                      jnp.float32),
        compiler_params=pltpu.CompilerParams(use_tc_tiling_on_sc=False,
                                             needs_layout_passes=False),
        scratch_types=[
            pltpu.VMEM((2, CL, 128), jnp.int32),
            pltpu.VMEM((2, CL, 128), jnp.int32),
            pltpu.VMEM((2, CL, 128, E), jnp.float32),
            pltpu.VMEM((CL, E // 8, 8, 128), jnp.float32),
            pltpu.SemaphoreType.DMA,
            pltpu.SemaphoreType.DMA,
            pltpu.SemaphoreType.DMA,
            pltpu.SemaphoreType.DMA,
        ],
    )
    def sc_kernel(seq_hbm, lab_hbm, tok_hbm, ps_hbm, out_hbm,
                  idx_v, psi_v, rows_v, tile_v, sem_t0, sem_p0, sem_t1, sem_p1):
        wid = lax.axis_index("s") * num_cores + lax.axis_index("c")
        bt = wid % n_bt
        lg = wid // n_bt
        col0 = bt * 128
        sem_t = (sem_t0, sem_t1)
        sem_p = (sem_p0, sem_p1)

        b16s = [jnp.arange(_LANES, dtype=jnp.int32) + bg * _LANES
                for bg in range(128 // _LANES)]

        def prep(s, c):
            l0 = lg * l_per_w + c * CL
            pltpu.sync_copy(
                seq_hbm.at[pl.ds(l0, CL), pl.ds(col0, 128)], idx_v.at[s])
            pltpu.sync_copy(
                lab_hbm.at[pl.ds(l0, CL), pl.ds(col0, 128)], psi_v.at[s])
            for li in range(CL):
                for bg in range(128 // _LANES):
                    sl = pl.ds(bg * _LANES, _LANES)
                    psi_v[s, li, sl] = psi_v[s, li, sl] + 2 * (l0 + li)
            for li in range(CL):
                pltpu.async_copy(tok_hbm.at[idx_v.at[s, li]],
                                 rows_v.at[s, li], sem_t[s])

        def process(s, c, prefetch):
            l0 = lg * l_per_w + c * CL
            for li in range(CL):
                pltpu.make_async_copy(tok_hbm.at[idx_v.at[s, li]],
                                      rows_v.at[s, li], sem_t[s]).wait()
            for li in range(CL):
                pltpu.async_copy(ps_hbm.at[psi_v.at[s, li]],
                                 rows_v.at[s, li], sem_p[s], add=True)
            for li in range(CL):
                pltpu.make_async_copy(ps_hbm.at[psi_v.at[s, li]],
                                      rows_v.at[s, li], sem_p[s]).wait()

            def diag_body(k, _):
                perm = (jnp.arange(_LANES, dtype=jnp.int32) + k) & (_LANES - 1)
                e37 = [((e0 + perm) >> 3, (e0 + perm) & 7, e0 + perm)
                       for e0 in range(0, E, _LANES)]
                for li in range(CL):
                    for b16 in b16s:
                        vals = [
                            plsc.load_gather(rows_v.at[s, li], [b16, ev])
                            for (_, _, ev) in e37
                        ]
                        for (e3v, e7v, _), val in zip(e37, vals):
                            plsc.store_scatter(tile_v.at[li], [e3v, e7v, b16],
                                               val)
                return 0

            lax.fori_loop(0, _LANES, diag_body, 0)
            if prefetch:
                @pl.when(c + 2 < n_chunks)
                def _():
                    prep(s, c + 2)
            for li in range(CL):
                pltpu.sync_copy(tile_v.at[li], out_hbm.at[l0 + li, :, bt])

        prep(0, 0)
        prep(1, 1)

        def pair_body(p, _):
            process(0, 2 * p, True)
            process(1, 2 * p + 1, True)
            return 0

        lax.fori_loop(0, n_chunks // 2, pair_body, 0)
        if n_chunks % 2:
            process(0, n_chunks - 1, False)

    return sc_kernel


def kernel(sequence, label, token_table, position_table, segment_table):
    B, L = sequence.shape
    V, E = token_table.shape
    ps_table = (position_table[:L, None, :]
                + segment_table[None, :2, :]).reshape(2 * L, E)
    CL = 2
    sc = _build_sc_kernel(B, L, E, CL)
    out5 = sc(sequence.T, label.T, token_table, ps_table)
    return out5.transpose((2, 4, 0, 1, 3)).reshape(B, L, E)


# speedup vs baseline: 1.2926x; 1.1442x over previous
import functools

import jax
import jax.numpy as jnp
from jax import lax
from jax.experimental import pallas as pl
from jax.experimental.pallas import tpu as pltpu
from jax.experimental.pallas import tpu_sc as plsc

_LANES = 16


def _build_sc_kernel(B, L, E, n_workers, chunk):
    N = B * L
    n_chunks = N // (n_workers * chunk)
    per_w = N // n_workers
    seq_per_chunk = chunk // L
    mesh = plsc.VectorSubcoreMesh(core_axis_name="c", subcore_axis_name="s")
    num_cores = plsc.get_sparse_core_info().num_cores

    @functools.partial(
        pl.kernel,
        mesh=mesh,
        out_type=jax.ShapeDtypeStruct((B, L, E), jnp.float32),
        compiler_params=pltpu.CompilerParams(use_tc_tiling_on_sc=False),
        scratch_types=[
            pltpu.VMEM((2, chunk), jnp.int32),
            pltpu.VMEM((2, chunk), jnp.int32),
            pltpu.VMEM((2, chunk, E), jnp.float32),
            pltpu.SemaphoreType.DMA,
            pltpu.SemaphoreType.DMA,
            pltpu.SemaphoreType.DMA,
            pltpu.SemaphoreType.DMA,
        ],
    )
    def sc_kernel(seq_hbm, lab_hbm, tok_hbm, ps_hbm, out_hbm,
                  idx_v, psi_v, tok_v, sem_t0, sem_p0, sem_t1, sem_p1):
        wid = lax.axis_index("s") * num_cores + lax.axis_index("c")
        base = wid * per_w
        sems = ((sem_t0, sem_p0), (sem_t1, sem_p1))
        iota = jnp.arange(_LANES, dtype=jnp.int32)

        def prep(c):
            s = c % 2
            row0 = base + c * chunk
            idx = idx_v.at[s]
            psi = psi_v.at[s]
            pltpu.sync_copy(seq_hbm.at[pl.ds(row0, chunk)], idx)
            pltpu.sync_copy(lab_hbm.at[pl.ds(row0, chunk)], psi)
            for j in range(chunk // _LANES):
                sl = pl.ds(j * _LANES, _LANES)
                psi[sl] = ((iota + (j * _LANES) % L) % L) * 2 + psi[sl]
            return pltpu.async_copy(tok_hbm.at[idx], tok_v.at[s], sems[s][0])

        pend = prep(0)
        for c in range(n_chunks):
            s = c % 2
            pend.wait()
            cp_ps = pltpu.async_copy(ps_hbm.at[psi_v.at[s]], tok_v.at[s],
                                     sems[s][1], add=True)
            nxt = prep(c + 1) if c + 1 < n_chunks else None
            cp_ps.wait()
            b0 = wid * (per_w // L) + c * seq_per_chunk
            for i in range(seq_per_chunk):
                pltpu.sync_copy(tok_v.at[s, pl.ds(i * L, L)], out_hbm.at[b0 + i])
            pend = nxt

    return sc_kernel


def kernel(sequence, label, token_table, position_table, segment_table):
    B, L = sequence.shape
    V, E = token_table.shape
    N = B * L
    ps_table = (position_table[:L, None, :]
                + segment_table[None, :2, :]).reshape(2 * L, E)
    n_workers = 32
    chunk = 400
    assert N % (n_workers * chunk) == 0 and chunk % L == 0
    sc = _build_sc_kernel(B, L, E, n_workers, chunk)
    return sc(sequence.reshape(N), label.reshape(N), token_table, ps_table)
